# baseline (speedup 1.0000x reference)
import functools

import jax
import jax.numpy as jnp
from jax import lax
from jax.experimental import pallas as pl
from jax.experimental.pallas import tpu as pltpu
from jax.experimental.pallas import tpu_sc as plsc

N_NODES = 10000
N_EDGES = 320000
FEAT = 128

NC = 2
NS = 16
NW = NC * NS
BATCH = 128
ROWS_PER_TILE = 80
IDX_HALF = ROWS_PER_TILE // 2
E_PAD = NW * ROWS_PER_TILE * BATCH
ACC_ROWS = 10112
OUT_PER_TILE = ACC_ROWS // NS


def _sc_body(
    x_hbm, src_hbm, dst_hbm, out_hbm, sidx, didx, rows_a, rows_b, accum,
    sem_a, sem_b,
):
    c = lax.axis_index("c")
    s = lax.axis_index("s")
    wid = s * NC + c

    def _zero_row(r, _):
        for cc in range(FEAT // 16):
            rows_a[r, pl.ds(cc * 16, 16)] = jnp.zeros((16,), jnp.float32)
        return 0

    lax.fori_loop(0, BATCH, _zero_row, 0)
    off = 0
    while off < OUT_PER_TILE:
        sz = min(BATCH, OUT_PER_TILE - off)
        pltpu.sync_copy(
            rows_a.at[pl.ds(0, sz)], accum.at[pl.ds(s * OUT_PER_TILE + off, sz)]
        )
        off += sz
    plsc.subcore_barrier()

    def _step_pair(p, _):
        j0 = 2 * p
        j1 = j0 + 1
        pltpu.make_async_copy(x_hbm.at[sidx.at[j0]], rows_a, sem_a).wait()
        pltpu.sync_copy(rows_a, accum.at[didx.at[j0]], add=True)
        pltpu.async_copy(x_hbm.at[sidx.at[j0 + 2]], rows_a, sem_a)
        pltpu.make_async_copy(x_hbm.at[sidx.at[j1]], rows_b, sem_b).wait()
        pltpu.sync_copy(rows_b, accum.at[didx.at[j1]], add=True)
        pltpu.async_copy(x_hbm.at[sidx.at[j1 + 2]], rows_b, sem_b)
        return 0

    for h in range(ROWS_PER_TILE // IDX_HALF):
        pltpu.sync_copy(src_hbm.at[wid, pl.ds(h * IDX_HALF, IDX_HALF)], sidx)
        pltpu.sync_copy(dst_hbm.at[wid, pl.ds(h * IDX_HALF, IDX_HALF)], didx)
        pltpu.async_copy(x_hbm.at[sidx.at[0]], rows_a, sem_a)
        pltpu.async_copy(x_hbm.at[sidx.at[1]], rows_b, sem_b)
        lax.fori_loop(0, IDX_HALF // 2 - 1, _step_pair, 0)
        jlast = IDX_HALF - 2
        pltpu.make_async_copy(x_hbm.at[sidx.at[jlast]], rows_a, sem_a).wait()
        pltpu.sync_copy(rows_a, accum.at[didx.at[jlast]], add=True)
        pltpu.make_async_copy(x_hbm.at[sidx.at[jlast + 1]], rows_b, sem_b).wait()
        pltpu.sync_copy(rows_b, accum.at[didx.at[jlast + 1]], add=True)
    plsc.subcore_barrier()

    pltpu.sync_copy(
        accum.at[pl.ds(s * OUT_PER_TILE, OUT_PER_TILE)],
        out_hbm.at[c, pl.ds(s * OUT_PER_TILE, OUT_PER_TILE)],
    )


@jax.jit
def _sc_spmm(x, src_r, dst_r):
    mesh = plsc.VectorSubcoreMesh(core_axis_name="c", subcore_axis_name="s")
    return pl.kernel(
        _sc_body,
        out_type=jax.ShapeDtypeStruct((NC, ACC_ROWS, FEAT), jnp.float32),
        mesh=mesh,
        scratch_types=[
            pltpu.VMEM((IDX_HALF, BATCH), jnp.int32),
            pltpu.VMEM((IDX_HALF, BATCH), jnp.int32),
            pltpu.VMEM((BATCH, FEAT), jnp.float32),
            pltpu.VMEM((BATCH, FEAT), jnp.float32),
            pltpu.VMEM_SHARED((ACC_ROWS, FEAT), jnp.float32),
            pltpu.SemaphoreType.DMA,
            pltpu.SemaphoreType.DMA,
        ],
    )(x, src_r, dst_r)


def _tc_body(p0_ref, p1_ref, w_ref, b_ref, o_ref):
    h = p0_ref[0] + p1_ref[0]
    o_ref[...] = (
        jnp.dot(h, w_ref[...], preferred_element_type=jnp.float32) + b_ref[...]
    )


@jax.jit
def _tc_combine(partial, W, b):
    blk = 1000
    grid = (N_NODES // blk,)
    return pl.pallas_call(
        _tc_body,
        grid=grid,
        in_specs=[
            pl.BlockSpec((1, blk, FEAT), lambda i: (0, i, 0)),
            pl.BlockSpec((1, blk, FEAT), lambda i: (1, i, 0)),
            pl.BlockSpec((FEAT, FEAT), lambda i: (0, 0)),
            pl.BlockSpec((1, FEAT), lambda i: (0, 0)),
        ],
        out_specs=pl.BlockSpec((blk, FEAT), lambda i: (i, 0)),
        out_shape=jax.ShapeDtypeStruct((N_NODES, FEAT), jnp.float32),
    )(partial, partial, W, b)


def kernel(x, edge_index, W, b):
    src = edge_index[0].astype(jnp.int32)
    dst = edge_index[1].astype(jnp.int32)
    pad = E_PAD - N_EDGES
    src_r = jnp.concatenate([src, jnp.zeros((pad,), jnp.int32)]).reshape(
        NW, ROWS_PER_TILE, BATCH
    )
    dst_r = jnp.concatenate(
        [dst, jnp.full((pad,), N_NODES, jnp.int32)]
    ).reshape(NW, ROWS_PER_TILE, BATCH)
    partial = _sc_spmm(x, src_r, dst_r)
    return _tc_combine(partial, W, b)


# speedup vs baseline: 1.2411x; 1.2411x over previous
import functools

import jax
import jax.numpy as jnp
from jax import lax
from jax.experimental import pallas as pl
from jax.experimental.pallas import tpu as pltpu
from jax.experimental.pallas import tpu_sc as plsc

N_NODES = 10000
N_EDGES = 320000
FEAT = 128

NC = 2
NS = 16
BATCH = 128
ROWS_C0 = 53
ROWS_C1 = 105
E_PAD = NS * (ROWS_C0 + ROWS_C1) * BATCH
ACC_ROWS = 10112
OUT_PER_TILE = ACC_ROWS // NS


def _sc_body(
    x_hbm, src0_hbm, dst0_hbm, src1_hbm, dst1_hbm, out_hbm, sidx, didx,
    rows, accum, sem,
):
    c = lax.axis_index("c")
    s = lax.axis_index("s")

    @pl.when(c == 0)
    def _stage0():
        pltpu.sync_copy(src0_hbm.at[s], sidx.at[pl.ds(0, ROWS_C0)])
        pltpu.sync_copy(dst0_hbm.at[s], didx.at[pl.ds(0, ROWS_C0)])

    @pl.when(c == 1)
    def _stage1():
        pltpu.sync_copy(src1_hbm.at[s], sidx.at[pl.ds(0, ROWS_C1)])
        pltpu.sync_copy(dst1_hbm.at[s], didx.at[pl.ds(0, ROWS_C1)])

    def _zero_row(r, _):
        for cc in range(FEAT // 16):
            rows[r, pl.ds(cc * 16, 16)] = jnp.zeros((16,), jnp.float32)
        return 0

    lax.fori_loop(0, BATCH, _zero_row, 0)
    off = 0
    while off < OUT_PER_TILE:
        sz = min(BATCH, OUT_PER_TILE - off)
        pltpu.sync_copy(
            rows.at[pl.ds(0, sz)], accum.at[pl.ds(s * OUT_PER_TILE + off, sz)]
        )
        off += sz
    plsc.subcore_barrier()

    def _step(j, _):
        pltpu.async_copy(x_hbm.at[sidx.at[j]], rows, sem).wait()
        pltpu.sync_copy(rows, accum.at[didx.at[j]], add=True)
        return 0

    @pl.when(c == 0)
    def _loop0():
        lax.fori_loop(0, ROWS_C0, _step, 0)

    @pl.when(c == 1)
    def _loop1():
        lax.fori_loop(0, ROWS_C1, _step, 0)

    plsc.subcore_barrier()

    pltpu.sync_copy(
        accum.at[pl.ds(s * OUT_PER_TILE, OUT_PER_TILE)],
        out_hbm.at[c, pl.ds(s * OUT_PER_TILE, OUT_PER_TILE)],
    )


@jax.jit
def _sc_spmm(x, src0, dst0, src1, dst1):
    mesh = plsc.VectorSubcoreMesh(core_axis_name="c", subcore_axis_name="s")
    return pl.kernel(
        _sc_body,
        out_type=jax.ShapeDtypeStruct((NC, ACC_ROWS, FEAT), jnp.float32),
        mesh=mesh,
        scratch_types=[
            pltpu.VMEM((ROWS_C1, BATCH), jnp.int32),
            pltpu.VMEM((ROWS_C1, BATCH), jnp.int32),
            pltpu.VMEM((BATCH, FEAT), jnp.float32),
            pltpu.VMEM_SHARED((ACC_ROWS, FEAT), jnp.float32),
            pltpu.SemaphoreType.DMA,
        ],
    )(x, src0, dst0, src1, dst1)


def _tc_body(p0_ref, p1_ref, w_ref, b_ref, o_ref):
    h = p0_ref[0] + p1_ref[0]
    o_ref[...] = (
        jnp.dot(h, w_ref[...], preferred_element_type=jnp.float32) + b_ref[...]
    )


@jax.jit
def _tc_combine(partial, W, b):
    blk = 1000
    grid = (N_NODES // blk,)
    return pl.pallas_call(
        _tc_body,
        grid=grid,
        in_specs=[
            pl.BlockSpec((1, blk, FEAT), lambda i: (0, i, 0)),
            pl.BlockSpec((1, blk, FEAT), lambda i: (1, i, 0)),
            pl.BlockSpec((FEAT, FEAT), lambda i: (0, 0)),
            pl.BlockSpec((1, FEAT), lambda i: (0, 0)),
        ],
        out_specs=pl.BlockSpec((blk, FEAT), lambda i: (i, 0)),
        out_shape=jax.ShapeDtypeStruct((N_NODES, FEAT), jnp.float32),
    )(partial, partial, W, b)


def kernel(x, edge_index, W, b):
    src = edge_index[0].astype(jnp.int32)
    dst = edge_index[1].astype(jnp.int32)
    pad = E_PAD - N_EDGES
    src_p = jnp.concatenate([src, jnp.zeros((pad,), jnp.int32)])
    dst_p = jnp.concatenate([dst, jnp.full((pad,), N_NODES, jnp.int32)])
    n0 = NS * ROWS_C0 * BATCH
    src0 = src_p[:n0].reshape(NS, ROWS_C0, BATCH)
    dst0 = dst_p[:n0].reshape(NS, ROWS_C0, BATCH)
    src1 = src_p[n0:].reshape(NS, ROWS_C1, BATCH)
    dst1 = dst_p[n0:].reshape(NS, ROWS_C1, BATCH)
    partial = _sc_spmm(x, src0, dst0, src1, dst1)
    return _tc_combine(partial, W, b)


# speedup vs baseline: 1.5555x; 1.2533x over previous
import functools

import jax
import jax.numpy as jnp
from jax import lax
from jax.experimental import pallas as pl
from jax.experimental.pallas import tpu as pltpu
from jax.experimental.pallas import tpu_sc as plsc

N_NODES = 10000
N_EDGES = 320000
FEAT = 128

NC = 2
NS = 16
BATCH = 128
ROWS_C0 = 105
ROWS_C1 = 53
E_PAD = NS * (ROWS_C0 + ROWS_C1) * BATCH
ACC_ROWS = 10112
OUT_PER_TILE = ACC_ROWS // NS


def _sc_body(
    x_hbm, src0_hbm, dst0_hbm, src1_hbm, dst1_hbm, out_hbm, sidx, didx,
    rows, accum, sem,
):
    c = lax.axis_index("c")
    s = lax.axis_index("s")

    @pl.when(c == 0)
    def _stage0():
        pltpu.sync_copy(src0_hbm.at[s], sidx.at[pl.ds(0, ROWS_C0)])
        pltpu.sync_copy(dst0_hbm.at[s], didx.at[pl.ds(0, ROWS_C0)])

    @pl.when(c == 1)
    def _stage1():
        pltpu.sync_copy(src1_hbm.at[s], sidx.at[pl.ds(0, ROWS_C1)])
        pltpu.sync_copy(dst1_hbm.at[s], didx.at[pl.ds(0, ROWS_C1)])

    def _zero_row(r, _):
        for cc in range(FEAT // 16):
            rows[r, pl.ds(cc * 16, 16)] = jnp.zeros((16,), jnp.float32)
        return 0

    lax.fori_loop(0, BATCH, _zero_row, 0)
    off = 0
    while off < OUT_PER_TILE:
        sz = min(BATCH, OUT_PER_TILE - off)
        pltpu.sync_copy(
            rows.at[pl.ds(0, sz)], accum.at[pl.ds(s * OUT_PER_TILE + off, sz)]
        )
        off += sz
    plsc.subcore_barrier()

    def _step(j, _):
        pltpu.async_copy(x_hbm.at[sidx.at[j]], rows, sem).wait()
        pltpu.sync_copy(rows, accum.at[didx.at[j]], add=True)
        return 0

    @pl.when(c == 0)
    def _loop0():
        lax.fori_loop(0, ROWS_C0, _step, 0)

    @pl.when(c == 1)
    def _loop1():
        lax.fori_loop(0, ROWS_C1, _step, 0)

    plsc.subcore_barrier()

    pltpu.sync_copy(
        accum.at[pl.ds(s * OUT_PER_TILE, OUT_PER_TILE)],
        out_hbm.at[c, pl.ds(s * OUT_PER_TILE, OUT_PER_TILE)],
    )


@jax.jit
def _sc_spmm(x, src0, dst0, src1, dst1):
    mesh = plsc.VectorSubcoreMesh(core_axis_name="c", subcore_axis_name="s")
    return pl.kernel(
        _sc_body,
        out_type=jax.ShapeDtypeStruct((NC, ACC_ROWS, FEAT), jnp.float32),
        mesh=mesh,
        scratch_types=[
            pltpu.VMEM((max(ROWS_C0, ROWS_C1), BATCH), jnp.int32),
            pltpu.VMEM((max(ROWS_C0, ROWS_C1), BATCH), jnp.int32),
            pltpu.VMEM((BATCH, FEAT), jnp.float32),
            pltpu.VMEM_SHARED((ACC_ROWS, FEAT), jnp.float32),
            pltpu.SemaphoreType.DMA,
        ],
    )(x, src0, dst0, src1, dst1)


def _tc_body(p0_ref, p1_ref, w_ref, b_ref, o_ref):
    h = p0_ref[0] + p1_ref[0]
    o_ref[...] = (
        jnp.dot(h, w_ref[...], preferred_element_type=jnp.float32) + b_ref[...]
    )


@jax.jit
def _tc_combine(partial, W, b):
    blk = 1000
    grid = (N_NODES // blk,)
    return pl.pallas_call(
        _tc_body,
        grid=grid,
        in_specs=[
            pl.BlockSpec((1, blk, FEAT), lambda i: (0, i, 0)),
            pl.BlockSpec((1, blk, FEAT), lambda i: (1, i, 0)),
            pl.BlockSpec((FEAT, FEAT), lambda i: (0, 0)),
            pl.BlockSpec((1, FEAT), lambda i: (0, 0)),
        ],
        out_specs=pl.BlockSpec((blk, FEAT), lambda i: (i, 0)),
        out_shape=jax.ShapeDtypeStruct((N_NODES, FEAT), jnp.float32),
    )(partial, partial, W, b)


def kernel(x, edge_index, W, b):
    src = edge_index[0].astype(jnp.int32)
    dst = edge_index[1].astype(jnp.int32)
    pad = E_PAD - N_EDGES
    src_p = jnp.concatenate([src, jnp.zeros((pad,), jnp.int32)])
    dst_p = jnp.concatenate([dst, jnp.full((pad,), N_NODES, jnp.int32)])
    n0 = NS * ROWS_C0 * BATCH
    src0 = src_p[:n0].reshape(NS, ROWS_C0, BATCH)
    dst0 = dst_p[:n0].reshape(NS, ROWS_C0, BATCH)
    src1 = src_p[n0:].reshape(NS, ROWS_C1, BATCH)
    dst1 = dst_p[n0:].reshape(NS, ROWS_C1, BATCH)
    partial = _sc_spmm(x, src0, dst0, src1, dst1)
    return _tc_combine(partial, W, b)


# speedup vs baseline: 1.7125x; 1.1010x over previous
import functools

import jax
import jax.numpy as jnp
from jax import lax
from jax.experimental import pallas as pl
from jax.experimental.pallas import tpu as pltpu
from jax.experimental.pallas import tpu_sc as plsc

N_NODES = 10000
N_EDGES = 320000
FEAT = 128

NC = 2
NS = 16
BATCH = 128
ROWS_C0 = 120
ROWS_C1 = 38
PHASE = 40
E_PAD = NS * (ROWS_C0 + ROWS_C1) * BATCH
ACC_ROWS = 10112
OUT_PER_TILE = ACC_ROWS // NS


def _sc_body(
    x_hbm, src0_hbm, dst0_hbm, src1_hbm, dst1_hbm, out_hbm, sidx, didx,
    rows_a, rows_b, accum, sem_a, sem_b,
):
    c = lax.axis_index("c")
    s = lax.axis_index("s")

    def _zero_row(r, _):
        for cc in range(FEAT // 16):
            rows_a[r, pl.ds(cc * 16, 16)] = jnp.zeros((16,), jnp.float32)
        return 0

    lax.fori_loop(0, BATCH, _zero_row, 0)
    off = 0
    while off < OUT_PER_TILE:
        sz = min(BATCH, OUT_PER_TILE - off)
        pltpu.sync_copy(
            rows_a.at[pl.ds(0, sz)], accum.at[pl.ds(s * OUT_PER_TILE + off, sz)]
        )
        off += sz
    plsc.subcore_barrier()

    def _step_pair(p, _):
        j0 = 2 * p
        j1 = j0 + 1
        pltpu.make_async_copy(x_hbm.at[sidx.at[j0]], rows_a, sem_a).wait()
        pltpu.sync_copy(rows_a, accum.at[didx.at[j0]], add=True)
        pltpu.async_copy(x_hbm.at[sidx.at[j0 + 2]], rows_a, sem_a)
        pltpu.make_async_copy(x_hbm.at[sidx.at[j1]], rows_b, sem_b).wait()
        pltpu.sync_copy(rows_b, accum.at[didx.at[j1]], add=True)
        pltpu.async_copy(x_hbm.at[sidx.at[j1 + 2]], rows_b, sem_b)
        return 0

    def _step_serial(j, _):
        pltpu.async_copy(x_hbm.at[sidx.at[j]], rows_a, sem_a).wait()
        pltpu.sync_copy(rows_a, accum.at[didx.at[j]], add=True)
        return 0

    @pl.when(c == 0)
    def _fast():
        for h in range(ROWS_C0 // PHASE):
            pltpu.sync_copy(
                src0_hbm.at[s, pl.ds(h * PHASE, PHASE)], sidx.at[pl.ds(0, PHASE)]
            )
            pltpu.sync_copy(
                dst0_hbm.at[s, pl.ds(h * PHASE, PHASE)], didx.at[pl.ds(0, PHASE)]
            )
            pltpu.async_copy(x_hbm.at[sidx.at[0]], rows_a, sem_a)
            pltpu.async_copy(x_hbm.at[sidx.at[1]], rows_b, sem_b)
            lax.fori_loop(0, PHASE // 2 - 1, _step_pair, 0)
            jlast = PHASE - 2
            pltpu.make_async_copy(x_hbm.at[sidx.at[jlast]], rows_a, sem_a).wait()
            pltpu.sync_copy(rows_a, accum.at[didx.at[jlast]], add=True)
            pltpu.make_async_copy(
                x_hbm.at[sidx.at[jlast + 1]], rows_b, sem_b
            ).wait()
            pltpu.sync_copy(rows_b, accum.at[didx.at[jlast + 1]], add=True)

    @pl.when(c == 1)
    def _slow():
        pltpu.sync_copy(src1_hbm.at[s], sidx.at[pl.ds(0, ROWS_C1)])
        pltpu.sync_copy(dst1_hbm.at[s], didx.at[pl.ds(0, ROWS_C1)])
        lax.fori_loop(0, ROWS_C1, _step_serial, 0)

    plsc.subcore_barrier()

    pltpu.sync_copy(
        accum.at[pl.ds(s * OUT_PER_TILE, OUT_PER_TILE)],
        out_hbm.at[c, pl.ds(s * OUT_PER_TILE, OUT_PER_TILE)],
    )


@jax.jit
def _sc_spmm(x, src0, dst0, src1, dst1):
    mesh = plsc.VectorSubcoreMesh(core_axis_name="c", subcore_axis_name="s")
    return pl.kernel(
        _sc_body,
        out_type=jax.ShapeDtypeStruct((NC, ACC_ROWS, FEAT), jnp.float32),
        mesh=mesh,
        scratch_types=[
            pltpu.VMEM((PHASE, BATCH), jnp.int32),
            pltpu.VMEM((PHASE, BATCH), jnp.int32),
            pltpu.VMEM((BATCH, FEAT), jnp.float32),
            pltpu.VMEM((BATCH, FEAT), jnp.float32),
            pltpu.VMEM_SHARED((ACC_ROWS, FEAT), jnp.float32),
            pltpu.SemaphoreType.DMA,
            pltpu.SemaphoreType.DMA,
        ],
    )(x, src0, dst0, src1, dst1)


def _tc_body(p0_ref, p1_ref, w_ref, b_ref, o_ref):
    h = p0_ref[0] + p1_ref[0]
    o_ref[...] = (
        jnp.dot(h, w_ref[...], preferred_element_type=jnp.float32) + b_ref[...]
    )


@jax.jit
def _tc_combine(partial, W, b):
    blk = 1000
    grid = (N_NODES // blk,)
    return pl.pallas_call(
        _tc_body,
        grid=grid,
        in_specs=[
            pl.BlockSpec((1, blk, FEAT), lambda i: (0, i, 0)),
            pl.BlockSpec((1, blk, FEAT), lambda i: (1, i, 0)),
            pl.BlockSpec((FEAT, FEAT), lambda i: (0, 0)),
            pl.BlockSpec((1, FEAT), lambda i: (0, 0)),
        ],
        out_specs=pl.BlockSpec((blk, FEAT), lambda i: (i, 0)),
        out_shape=jax.ShapeDtypeStruct((N_NODES, FEAT), jnp.float32),
    )(partial, partial, W, b)


def kernel(x, edge_index, W, b):
    src = edge_index[0].astype(jnp.int32)
    dst = edge_index[1].astype(jnp.int32)
    pad = E_PAD - N_EDGES
    src_p = jnp.concatenate([src, jnp.zeros((pad,), jnp.int32)])
    dst_p = jnp.concatenate([dst, jnp.full((pad,), N_NODES, jnp.int32)])
    n0 = NS * ROWS_C0 * BATCH
    src0 = src_p[:n0].reshape(NS, ROWS_C0, BATCH)
    dst0 = dst_p[:n0].reshape(NS, ROWS_C0, BATCH)
    src1 = src_p[n0:].reshape(NS, ROWS_C1, BATCH)
    dst1 = dst_p[n0:].reshape(NS, ROWS_C1, BATCH)
    partial = _sc_spmm(x, src0, dst0, src1, dst1)
    return _tc_combine(partial, W, b)


# speedup vs baseline: 1.8212x; 1.0635x over previous
import functools

import jax
import jax.numpy as jnp
from jax import lax
from jax.experimental import pallas as pl
from jax.experimental.pallas import tpu as pltpu
from jax.experimental.pallas import tpu_sc as plsc

N_NODES = 10000
N_EDGES = 320000
FEAT = 128

NC = 2
NS = 16
BATCH = 128
ROWS_C0 = 134
ROWS_C1 = 24
PHASE = 40
PHASES_C0 = (40, 40, 40, 14)
E_PAD = NS * (ROWS_C0 + ROWS_C1) * BATCH
ACC_ROWS = 10112
OUT_PER_TILE = ACC_ROWS // NS


def _sc_body(
    x_hbm, src0_hbm, dst0_hbm, src1_hbm, dst1_hbm, out_hbm, sidx, didx,
    rows_a, rows_b, accum, sem_a, sem_b,
):
    c = lax.axis_index("c")
    s = lax.axis_index("s")

    def _zero_row(r, _):
        for cc in range(FEAT // 16):
            rows_a[r, pl.ds(cc * 16, 16)] = jnp.zeros((16,), jnp.float32)
        return 0

    lax.fori_loop(0, BATCH, _zero_row, 0)
    off = 0
    while off < OUT_PER_TILE:
        sz = min(BATCH, OUT_PER_TILE - off)
        pltpu.sync_copy(
            rows_a.at[pl.ds(0, sz)], accum.at[pl.ds(s * OUT_PER_TILE + off, sz)]
        )
        off += sz
    plsc.subcore_barrier()

    def _step_pair(p, _):
        j0 = 2 * p
        j1 = j0 + 1
        pltpu.make_async_copy(x_hbm.at[sidx.at[j0]], rows_a, sem_a).wait()
        pltpu.sync_copy(rows_a, accum.at[didx.at[j0]], add=True)
        pltpu.async_copy(x_hbm.at[sidx.at[j0 + 2]], rows_a, sem_a)
        pltpu.make_async_copy(x_hbm.at[sidx.at[j1]], rows_b, sem_b).wait()
        pltpu.sync_copy(rows_b, accum.at[didx.at[j1]], add=True)
        pltpu.async_copy(x_hbm.at[sidx.at[j1 + 2]], rows_b, sem_b)
        return 0

    def _step_serial(j, _):
        pltpu.async_copy(x_hbm.at[sidx.at[j]], rows_a, sem_a).wait()
        pltpu.sync_copy(rows_a, accum.at[didx.at[j]], add=True)
        return 0

    @pl.when(c == 0)
    def _fast():
        base = 0
        for ph_n in PHASES_C0:
            pltpu.sync_copy(
                src0_hbm.at[s, pl.ds(base, ph_n)], sidx.at[pl.ds(0, ph_n)]
            )
            pltpu.sync_copy(
                dst0_hbm.at[s, pl.ds(base, ph_n)], didx.at[pl.ds(0, ph_n)]
            )
            pltpu.async_copy(x_hbm.at[sidx.at[0]], rows_a, sem_a)
            pltpu.async_copy(x_hbm.at[sidx.at[1]], rows_b, sem_b)
            lax.fori_loop(0, ph_n // 2 - 1, _step_pair, 0)
            jlast = ph_n - 2
            pltpu.make_async_copy(x_hbm.at[sidx.at[jlast]], rows_a, sem_a).wait()
            pltpu.sync_copy(rows_a, accum.at[didx.at[jlast]], add=True)
            pltpu.make_async_copy(
                x_hbm.at[sidx.at[jlast + 1]], rows_b, sem_b
            ).wait()
            pltpu.sync_copy(rows_b, accum.at[didx.at[jlast + 1]], add=True)
            base += ph_n

    @pl.when(c == 1)
    def _slow():
        pltpu.sync_copy(src1_hbm.at[s], sidx.at[pl.ds(0, ROWS_C1)])
        pltpu.sync_copy(dst1_hbm.at[s], didx.at[pl.ds(0, ROWS_C1)])
        lax.fori_loop(0, ROWS_C1, _step_serial, 0)

    plsc.subcore_barrier()

    pltpu.sync_copy(
        accum.at[pl.ds(s * OUT_PER_TILE, OUT_PER_TILE)],
        out_hbm.at[c, pl.ds(s * OUT_PER_TILE, OUT_PER_TILE)],
    )


@jax.jit
def _sc_spmm(x, src0, dst0, src1, dst1):
    mesh = plsc.VectorSubcoreMesh(core_axis_name="c", subcore_axis_name="s")
    return pl.kernel(
        _sc_body,
        out_type=jax.ShapeDtypeStruct((NC, ACC_ROWS, FEAT), jnp.float32),
        mesh=mesh,
        scratch_types=[
            pltpu.VMEM((PHASE, BATCH), jnp.int32),
            pltpu.VMEM((PHASE, BATCH), jnp.int32),
            pltpu.VMEM((BATCH, FEAT), jnp.float32),
            pltpu.VMEM((BATCH, FEAT), jnp.float32),
            pltpu.VMEM_SHARED((ACC_ROWS, FEAT), jnp.float32),
            pltpu.SemaphoreType.DMA,
            pltpu.SemaphoreType.DMA,
        ],
    )(x, src0, dst0, src1, dst1)


def _tc_body(p0_ref, p1_ref, w_ref, b_ref, o_ref):
    h = p0_ref[0] + p1_ref[0]
    o_ref[...] = (
        jnp.dot(h, w_ref[...], preferred_element_type=jnp.float32) + b_ref[...]
    )


@jax.jit
def _tc_combine(partial, W, b):
    blk = 1000
    grid = (N_NODES // blk,)
    return pl.pallas_call(
        _tc_body,
        grid=grid,
        in_specs=[
            pl.BlockSpec((1, blk, FEAT), lambda i: (0, i, 0)),
            pl.BlockSpec((1, blk, FEAT), lambda i: (1, i, 0)),
            pl.BlockSpec((FEAT, FEAT), lambda i: (0, 0)),
            pl.BlockSpec((1, FEAT), lambda i: (0, 0)),
        ],
        out_specs=pl.BlockSpec((blk, FEAT), lambda i: (i, 0)),
        out_shape=jax.ShapeDtypeStruct((N_NODES, FEAT), jnp.float32),
    )(partial, partial, W, b)


def kernel(x, edge_index, W, b):
    src = edge_index[0].astype(jnp.int32)
    dst = edge_index[1].astype(jnp.int32)
    pad = E_PAD - N_EDGES
    src_p = jnp.concatenate([src, jnp.zeros((pad,), jnp.int32)])
    dst_p = jnp.concatenate([dst, jnp.full((pad,), N_NODES, jnp.int32)])
    n0 = NS * ROWS_C0 * BATCH
    src0 = src_p[:n0].reshape(NS, ROWS_C0, BATCH)
    dst0 = dst_p[:n0].reshape(NS, ROWS_C0, BATCH)
    src1 = src_p[n0:].reshape(NS, ROWS_C1, BATCH)
    dst1 = dst_p[n0:].reshape(NS, ROWS_C1, BATCH)
    partial = _sc_spmm(x, src0, dst0, src1, dst1)
    return _tc_combine(partial, W, b)


# speedup vs baseline: 1.9045x; 1.0458x over previous
import functools

import jax
import jax.numpy as jnp
from jax import lax
from jax.experimental import pallas as pl
from jax.experimental.pallas import tpu as pltpu
from jax.experimental.pallas import tpu_sc as plsc

N_NODES = 10000
N_EDGES = 320000
FEAT = 128

NC = 2
NS = 16
BATCH = 128
ROWS_C0 = 140
ROWS_C1 = 18
PHASE = 40
PHASES_C0 = (40, 40, 40, 20)
E_PAD = NS * (ROWS_C0 + ROWS_C1) * BATCH
ACC_ROWS = 10112
OUT_PER_TILE = ACC_ROWS // NS


def _sc_body(
    x_hbm, src0_hbm, dst0_hbm, src1_hbm, dst1_hbm, out_hbm, sidx, didx,
    rows_a, rows_b, accum, sem_a, sem_b,
):
    c = lax.axis_index("c")
    s = lax.axis_index("s")

    def _zero_row(r, _):
        for cc in range(FEAT // 16):
            rows_a[r, pl.ds(cc * 16, 16)] = jnp.zeros((16,), jnp.float32)
        return 0

    lax.fori_loop(0, BATCH, _zero_row, 0)
    off = 0
    while off < OUT_PER_TILE:
        sz = min(BATCH, OUT_PER_TILE - off)
        pltpu.sync_copy(
            rows_a.at[pl.ds(0, sz)], accum.at[pl.ds(s * OUT_PER_TILE + off, sz)]
        )
        off += sz
    plsc.subcore_barrier()

    def _step_pair(p, _):
        j0 = 2 * p
        j1 = j0 + 1
        pltpu.make_async_copy(x_hbm.at[sidx.at[j0]], rows_a, sem_a).wait()
        pltpu.sync_copy(rows_a, accum.at[didx.at[j0]], add=True)
        pltpu.async_copy(x_hbm.at[sidx.at[j0 + 2]], rows_a, sem_a)
        pltpu.make_async_copy(x_hbm.at[sidx.at[j1]], rows_b, sem_b).wait()
        pltpu.sync_copy(rows_b, accum.at[didx.at[j1]], add=True)
        pltpu.async_copy(x_hbm.at[sidx.at[j1 + 2]], rows_b, sem_b)
        return 0

    def _step_serial(j, _):
        pltpu.async_copy(x_hbm.at[sidx.at[j]], rows_a, sem_a).wait()
        pltpu.sync_copy(rows_a, accum.at[didx.at[j]], add=True)
        return 0

    @pl.when(c == 0)
    def _fast():
        base = 0
        for ph_n in PHASES_C0:
            pltpu.sync_copy(
                src0_hbm.at[s, pl.ds(base, ph_n)], sidx.at[pl.ds(0, ph_n)]
            )
            pltpu.sync_copy(
                dst0_hbm.at[s, pl.ds(base, ph_n)], didx.at[pl.ds(0, ph_n)]
            )
            pltpu.async_copy(x_hbm.at[sidx.at[0]], rows_a, sem_a)
            pltpu.async_copy(x_hbm.at[sidx.at[1]], rows_b, sem_b)
            lax.fori_loop(0, ph_n // 2 - 1, _step_pair, 0)
            jlast = ph_n - 2
            pltpu.make_async_copy(x_hbm.at[sidx.at[jlast]], rows_a, sem_a).wait()
            pltpu.sync_copy(rows_a, accum.at[didx.at[jlast]], add=True)
            pltpu.make_async_copy(
                x_hbm.at[sidx.at[jlast + 1]], rows_b, sem_b
            ).wait()
            pltpu.sync_copy(rows_b, accum.at[didx.at[jlast + 1]], add=True)
            base += ph_n

    @pl.when(c == 1)
    def _slow():
        pltpu.sync_copy(src1_hbm.at[s], sidx.at[pl.ds(0, ROWS_C1)])
        pltpu.sync_copy(dst1_hbm.at[s], didx.at[pl.ds(0, ROWS_C1)])
        lax.fori_loop(0, ROWS_C1, _step_serial, 0)

    plsc.subcore_barrier()

    pltpu.sync_copy(
        accum.at[pl.ds(s * OUT_PER_TILE, OUT_PER_TILE)],
        out_hbm.at[c, pl.ds(s * OUT_PER_TILE, OUT_PER_TILE)],
    )


@jax.jit
def _sc_spmm(x, src0, dst0, src1, dst1):
    mesh = plsc.VectorSubcoreMesh(core_axis_name="c", subcore_axis_name="s")
    return pl.kernel(
        _sc_body,
        out_type=jax.ShapeDtypeStruct((NC, ACC_ROWS, FEAT), jnp.float32),
        mesh=mesh,
        scratch_types=[
            pltpu.VMEM((PHASE, BATCH), jnp.int32),
            pltpu.VMEM((PHASE, BATCH), jnp.int32),
            pltpu.VMEM((BATCH, FEAT), jnp.float32),
            pltpu.VMEM((BATCH, FEAT), jnp.float32),
            pltpu.VMEM_SHARED((ACC_ROWS, FEAT), jnp.float32),
            pltpu.SemaphoreType.DMA,
            pltpu.SemaphoreType.DMA,
        ],
    )(x, src0, dst0, src1, dst1)


def _tc_body(p0_ref, p1_ref, w_ref, b_ref, o_ref):
    h = p0_ref[0] + p1_ref[0]
    o_ref[...] = (
        jnp.dot(h, w_ref[...], preferred_element_type=jnp.float32) + b_ref[...]
    )


@jax.jit
def _tc_combine(partial, W, b):
    blk = 1000
    grid = (N_NODES // blk,)
    return pl.pallas_call(
        _tc_body,
        grid=grid,
        in_specs=[
            pl.BlockSpec((1, blk, FEAT), lambda i: (0, i, 0)),
            pl.BlockSpec((1, blk, FEAT), lambda i: (1, i, 0)),
            pl.BlockSpec((FEAT, FEAT), lambda i: (0, 0)),
            pl.BlockSpec((1, FEAT), lambda i: (0, 0)),
        ],
        out_specs=pl.BlockSpec((blk, FEAT), lambda i: (i, 0)),
        out_shape=jax.ShapeDtypeStruct((N_NODES, FEAT), jnp.float32),
    )(partial, partial, W, b)


def kernel(x, edge_index, W, b):
    src = edge_index[0].astype(jnp.int32)
    dst = edge_index[1].astype(jnp.int32)
    pad = E_PAD - N_EDGES
    src_p = jnp.concatenate([src, jnp.zeros((pad,), jnp.int32)])
    dst_p = jnp.concatenate([dst, jnp.full((pad,), N_NODES, jnp.int32)])
    n0 = NS * ROWS_C0 * BATCH
    src0 = src_p[:n0].reshape(NS, ROWS_C0, BATCH)
    dst0 = dst_p[:n0].reshape(NS, ROWS_C0, BATCH)
    src1 = src_p[n0:].reshape(NS, ROWS_C1, BATCH)
    dst1 = dst_p[n0:].reshape(NS, ROWS_C1, BATCH)
    partial = _sc_spmm(x, src0, dst0, src1, dst1)
    return _tc_combine(partial, W, b)


# speedup vs baseline: 2.0616x; 1.0825x over previous
import functools

import jax
import jax.numpy as jnp
from jax import lax
from jax.experimental import pallas as pl
from jax.experimental.pallas import tpu as pltpu
from jax.experimental.pallas import tpu_sc as plsc

N_NODES = 10000
N_EDGES = 320000
FEAT = 128

NC = 2
NS = 16
BATCH = 128
ROWS_C0 = 146
ROWS_C1 = 12
PHASE = 40
PHASES_C0 = (40, 40, 40, 26)
E_PAD = NS * (ROWS_C0 + ROWS_C1) * BATCH
ACC_ROWS = 10112
OUT_PER_TILE = ACC_ROWS // NS


def _sc_body(
    x_hbm, src0_hbm, dst0_hbm, src1_hbm, dst1_hbm, out_hbm, sidx, didx,
    rows_a, rows_b, accum, sem_a, sem_b,
):
    c = lax.axis_index("c")
    s = lax.axis_index("s")

    def _zero_row(r, _):
        for cc in range(FEAT // 16):
            rows_a[r, pl.ds(cc * 16, 16)] = jnp.zeros((16,), jnp.float32)
        return 0

    lax.fori_loop(0, BATCH, _zero_row, 0)
    off = 0
    while off < OUT_PER_TILE:
        sz = min(BATCH, OUT_PER_TILE - off)
        pltpu.sync_copy(
            rows_a.at[pl.ds(0, sz)], accum.at[pl.ds(s * OUT_PER_TILE + off, sz)]
        )
        off += sz
    plsc.subcore_barrier()

    def _step_pair(p, _):
        j0 = 2 * p
        j1 = j0 + 1
        pltpu.make_async_copy(x_hbm.at[sidx.at[j0]], rows_a, sem_a).wait()
        pltpu.sync_copy(rows_a, accum.at[didx.at[j0]], add=True)
        pltpu.async_copy(x_hbm.at[sidx.at[j0 + 2]], rows_a, sem_a)
        pltpu.make_async_copy(x_hbm.at[sidx.at[j1]], rows_b, sem_b).wait()
        pltpu.sync_copy(rows_b, accum.at[didx.at[j1]], add=True)
        pltpu.async_copy(x_hbm.at[sidx.at[j1 + 2]], rows_b, sem_b)
        return 0

    def _step_serial(j, _):
        pltpu.async_copy(x_hbm.at[sidx.at[j]], rows_a, sem_a).wait()
        pltpu.sync_copy(rows_a, accum.at[didx.at[j]], add=True)
        return 0

    @pl.when(c == 0)
    def _fast():
        base = 0
        for ph_n in PHASES_C0:
            pltpu.sync_copy(
                src0_hbm.at[s, pl.ds(base, ph_n)], sidx.at[pl.ds(0, ph_n)]
            )
            pltpu.sync_copy(
                dst0_hbm.at[s, pl.ds(base, ph_n)], didx.at[pl.ds(0, ph_n)]
            )
            pltpu.async_copy(x_hbm.at[sidx.at[0]], rows_a, sem_a)
            pltpu.async_copy(x_hbm.at[sidx.at[1]], rows_b, sem_b)
            lax.fori_loop(0, ph_n // 2 - 1, _step_pair, 0)
            jlast = ph_n - 2
            pltpu.make_async_copy(x_hbm.at[sidx.at[jlast]], rows_a, sem_a).wait()
            pltpu.sync_copy(rows_a, accum.at[didx.at[jlast]], add=True)
            pltpu.make_async_copy(
                x_hbm.at[sidx.at[jlast + 1]], rows_b, sem_b
            ).wait()
            pltpu.sync_copy(rows_b, accum.at[didx.at[jlast + 1]], add=True)
            base += ph_n

    @pl.when(c == 1)
    def _slow():
        pltpu.sync_copy(src1_hbm.at[s], sidx.at[pl.ds(0, ROWS_C1)])
        pltpu.sync_copy(dst1_hbm.at[s], didx.at[pl.ds(0, ROWS_C1)])
        lax.fori_loop(0, ROWS_C1, _step_serial, 0)

    plsc.subcore_barrier()

    pltpu.sync_copy(
        accum.at[pl.ds(s * OUT_PER_TILE, OUT_PER_TILE)],
        out_hbm.at[c, pl.ds(s * OUT_PER_TILE, OUT_PER_TILE)],
    )


@jax.jit
def _sc_spmm(x, src0, dst0, src1, dst1):
    mesh = plsc.VectorSubcoreMesh(core_axis_name="c", subcore_axis_name="s")
    return pl.kernel(
        _sc_body,
        out_type=jax.ShapeDtypeStruct((NC, ACC_ROWS, FEAT), jnp.float32),
        mesh=mesh,
        scratch_types=[
            pltpu.VMEM((PHASE, BATCH), jnp.int32),
            pltpu.VMEM((PHASE, BATCH), jnp.int32),
            pltpu.VMEM((BATCH, FEAT), jnp.float32),
            pltpu.VMEM((BATCH, FEAT), jnp.float32),
            pltpu.VMEM_SHARED((ACC_ROWS, FEAT), jnp.float32),
            pltpu.SemaphoreType.DMA,
            pltpu.SemaphoreType.DMA,
        ],
    )(x, src0, dst0, src1, dst1)


def _tc_body(p0_ref, p1_ref, w_ref, b_ref, o_ref):
    h = p0_ref[0] + p1_ref[0]
    o_ref[...] = (
        jnp.dot(h, w_ref[...], preferred_element_type=jnp.float32) + b_ref[...]
    )


@jax.jit
def _tc_combine(partial, W, b):
    blk = 1000
    grid = (N_NODES // blk,)
    return pl.pallas_call(
        _tc_body,
        grid=grid,
        in_specs=[
            pl.BlockSpec((1, blk, FEAT), lambda i: (0, i, 0)),
            pl.BlockSpec((1, blk, FEAT), lambda i: (1, i, 0)),
            pl.BlockSpec((FEAT, FEAT), lambda i: (0, 0)),
            pl.BlockSpec((1, FEAT), lambda i: (0, 0)),
        ],
        out_specs=pl.BlockSpec((blk, FEAT), lambda i: (i, 0)),
        out_shape=jax.ShapeDtypeStruct((N_NODES, FEAT), jnp.float32),
    )(partial, partial, W, b)


def kernel(x, edge_index, W, b):
    src = edge_index[0].astype(jnp.int32)
    dst = edge_index[1].astype(jnp.int32)
    pad = E_PAD - N_EDGES
    src_p = jnp.concatenate([src, jnp.zeros((pad,), jnp.int32)])
    dst_p = jnp.concatenate([dst, jnp.full((pad,), N_NODES, jnp.int32)])
    n0 = NS * ROWS_C0 * BATCH
    src0 = src_p[:n0].reshape(NS, ROWS_C0, BATCH)
    dst0 = dst_p[:n0].reshape(NS, ROWS_C0, BATCH)
    src1 = src_p[n0:].reshape(NS, ROWS_C1, BATCH)
    dst1 = dst_p[n0:].reshape(NS, ROWS_C1, BATCH)
    partial = _sc_spmm(x, src0, dst0, src1, dst1)
    return _tc_combine(partial, W, b)
